# baseline (device time: 31874 ns/iter reference)
import jax
import jax.numpy as jnp
from jax import lax
from jax.experimental import pallas as pl
from jax.experimental.pallas import tpu as pltpu

N_DEV = 4
B, Sq, Skv, Dh = 2, 128, 128, 64
HL = 4
D_MODEL = 512
D_HEADS = HL * Dh
ROWS = B * Sq


def _body(x_ref, wq_ref, kf_ref, vf_ref, wo_ref, out_ref,
          ctx_ref, comm_ref, acc_ref, send_sems, recv_sems):
    my_i = lax.axis_index("i")
    left = lax.rem(my_i - 1 + N_DEV, N_DEV)
    right = lax.rem(my_i + 1, N_DEV)

    barrier_sem = pltpu.get_barrier_semaphore()
    for nbr in (left, right):
        pl.semaphore_signal(
            barrier_sem, inc=1,
            device_id=(nbr,), device_id_type=pl.DeviceIdType.MESH,
        )
    pl.semaphore_wait(barrier_sem, 2)

    qf = jnp.dot(x_ref[...], wq_ref[...],
                 preferred_element_type=jnp.float32)

    qb = lax.broadcasted_iota(jnp.int32, (Sq, Skv), 0) // 64
    kb = lax.broadcasted_iota(jnp.int32, (Sq, Skv), 1) // 64
    mask = kb <= qb

    for b in range(B):
        rs = slice(b * Sq, (b + 1) * Sq)
        for h in range(HL):
            cs = slice(h * Dh, (h + 1) * Dh)
            q = qf[rs, cs]
            k = kf_ref[rs, cs]
            v = vf_ref[rs, cs]
            s = lax.dot_general(
                q, k, (((1,), (1,)), ((), ())),
                preferred_element_type=jnp.float32) * 0.125
            s = jnp.where(mask, s, -1e9)
            m = jnp.max(s, axis=-1, keepdims=True)
            w = jnp.exp(s - m)
            w = w / jnp.sum(w, axis=-1, keepdims=True)
            ctx_ref[rs, cs] = jnp.dot(w, v,
                                      preferred_element_type=jnp.float32)

    partial = jnp.dot(ctx_ref[...], wo_ref[...],
                      preferred_element_type=jnp.float32)
    acc_ref[...] = partial
    comm_ref[0] = partial

    for h in range(N_DEV - 1):
        rdma = pltpu.make_async_remote_copy(
            src_ref=comm_ref.at[h],
            dst_ref=comm_ref.at[h + 1],
            send_sem=send_sems.at[h],
            recv_sem=recv_sems.at[h],
            device_id=(right,),
            device_id_type=pl.DeviceIdType.MESH,
        )
        rdma.start()
        rdma.wait()
        acc_ref[...] = acc_ref[...] + comm_ref[h + 1]

    out_ref[...] = acc_ref[...]


def kernel(x, Wq, K_ext, V_ext, Wo):
    my_i = lax.axis_index("i")
    Kh = lax.dynamic_slice_in_dim(K_ext, my_i * HL, HL, axis=2)
    Vh = lax.dynamic_slice_in_dim(V_ext, my_i * HL, HL, axis=2)

    out = pl.pallas_call(
        _body,
        out_shape=jax.ShapeDtypeStruct((ROWS, D_MODEL), jnp.float32),
        in_specs=[pl.BlockSpec(memory_space=pltpu.VMEM)] * 5,
        out_specs=pl.BlockSpec(memory_space=pltpu.VMEM),
        scratch_shapes=[
            pltpu.VMEM((ROWS, D_HEADS), jnp.float32),
            pltpu.VMEM((N_DEV, ROWS, D_MODEL), jnp.float32),
            pltpu.VMEM((ROWS, D_MODEL), jnp.float32),
            pltpu.SemaphoreType.DMA((N_DEV - 1,)),
            pltpu.SemaphoreType.DMA((N_DEV - 1,)),
        ],
        compiler_params=pltpu.CompilerParams(collective_id=0),
    )(
        x.reshape(ROWS, D_MODEL),
        Wq,
        Kh.reshape(B * Skv, D_HEADS),
        Vh.reshape(B * Skv, D_HEADS),
        Wo,
    )
    return out.reshape(B, Sq, D_MODEL)


# device time: 20281 ns/iter; 1.5716x vs baseline; 1.5716x over previous
import jax
import jax.numpy as jnp
from jax import lax
from jax.experimental import pallas as pl
from jax.experimental.pallas import tpu as pltpu

N_DEV = 4
B, Sq, Skv, Dh = 2, 128, 128, 64
HL = 4
D_MODEL = 512
D_HEADS = HL * Dh
ROWS = B * Sq


def _body(x_ref, wq_ref, kf_ref, vf_ref, wo_ref, out_ref,
          ctx_ref, r1_ref, r2_ref, send_sems, recv_sems):
    my_i = lax.axis_index("i")
    p1 = my_i ^ 1
    p2 = 3 - my_i

    barrier_sem = pltpu.get_barrier_semaphore()
    for nbr in (p1, p2):
        pl.semaphore_signal(
            barrier_sem, inc=1,
            device_id=(nbr,), device_id_type=pl.DeviceIdType.MESH,
        )
    pl.semaphore_wait(barrier_sem, 2)

    qf = jnp.dot(x_ref[...], wq_ref[...],
                 preferred_element_type=jnp.float32)

    qb = lax.broadcasted_iota(jnp.int32, (Sq, Skv), 0) // 64
    kb = lax.broadcasted_iota(jnp.int32, (Sq, Skv), 1) // 64
    mask = kb <= qb

    r1 = []
    for b in range(B):
        rs = pl.ds(b * Sq, Sq)
        for h in range(HL):
            cs = slice(h * Dh, (h + 1) * Dh)
            q = qf[b * Sq:(b + 1) * Sq, cs]
            k = kf_ref[rs, cs]
            v = vf_ref[rs, cs]
            s = lax.dot_general(
                q, k, (((1,), (1,)), ((), ())),
                preferred_element_type=jnp.float32) * 0.125
            s = jnp.where(mask, s, -1e9)
            m = jnp.max(s, axis=-1, keepdims=True)
            w = jnp.exp(s - m)
            w = w / jnp.sum(w, axis=-1, keepdims=True)
            ctx_ref[:, cs] = jnp.dot(w, v,
                                     preferred_element_type=jnp.float32)
        out_ref[rs, :] = jnp.dot(ctx_ref[...], wo_ref[...],
                                 preferred_element_type=jnp.float32)
        rdma = pltpu.make_async_remote_copy(
            src_ref=out_ref.at[rs],
            dst_ref=r1_ref.at[b],
            send_sem=send_sems.at[0, b],
            recv_sem=recv_sems.at[0, b],
            device_id=(p1,),
            device_id_type=pl.DeviceIdType.MESH,
        )
        rdma.start()
        r1.append(rdma)

    r2 = []
    for b in range(B):
        rs = pl.ds(b * Sq, Sq)
        r1[b].wait()
        out_ref[rs, :] = out_ref[rs, :] + r1_ref[b]
        rdma = pltpu.make_async_remote_copy(
            src_ref=out_ref.at[rs],
            dst_ref=r2_ref.at[b],
            send_sem=send_sems.at[1, b],
            recv_sem=recv_sems.at[1, b],
            device_id=(p2,),
            device_id_type=pl.DeviceIdType.MESH,
        )
        rdma.start()
        r2.append(rdma)

    for b in range(B):
        rs = pl.ds(b * Sq, Sq)
        r2[b].wait()
        out_ref[rs, :] = out_ref[rs, :] + r2_ref[b]


def kernel(x, Wq, K_ext, V_ext, Wo):
    my_i = lax.axis_index("i")
    Kh = lax.dynamic_slice_in_dim(K_ext, my_i * HL, HL, axis=2)
    Vh = lax.dynamic_slice_in_dim(V_ext, my_i * HL, HL, axis=2)

    out = pl.pallas_call(
        _body,
        out_shape=jax.ShapeDtypeStruct((ROWS, D_MODEL), jnp.float32),
        in_specs=[pl.BlockSpec(memory_space=pltpu.VMEM)] * 5,
        out_specs=pl.BlockSpec(memory_space=pltpu.VMEM),
        scratch_shapes=[
            pltpu.VMEM((Sq, D_HEADS), jnp.float32),
            pltpu.VMEM((B, Sq, D_MODEL), jnp.float32),
            pltpu.VMEM((B, Sq, D_MODEL), jnp.float32),
            pltpu.SemaphoreType.DMA((2, B)),
            pltpu.SemaphoreType.DMA((2, B)),
        ],
        compiler_params=pltpu.CompilerParams(collective_id=0),
    )(
        x.reshape(ROWS, D_MODEL),
        Wq,
        Kh.reshape(B * Skv, D_HEADS),
        Vh.reshape(B * Skv, D_HEADS),
        Wo,
    )
    return out.reshape(B, Sq, D_MODEL)


# device time: 17055 ns/iter; 1.8689x vs baseline; 1.1892x over previous
import jax
import jax.numpy as jnp
from jax import lax
from jax.experimental import pallas as pl
from jax.experimental.pallas import tpu as pltpu

N_DEV = 4
B, Sq, Skv, Dh = 2, 128, 128, 64
HL = 4
D_MODEL = 512
D_HEADS = HL * Dh
ROWS = B * Sq
BLK = 64


def _attn_block(q, k, v):
    s = lax.dot_general(q, k, (((1,), (1,)), ((), ())),
                        preferred_element_type=jnp.float32) * 0.125
    m = jnp.max(s, axis=-1, keepdims=True)
    w = jnp.exp(s - m)
    r = 1.0 / jnp.sum(w, axis=-1, keepdims=True)
    ctx = jnp.dot(w.astype(jnp.bfloat16), v,
                  preferred_element_type=jnp.float32)
    return ctx * r


def _body(x_ref, wq_ref, kf_ref, vf_ref, wo_ref, out_ref,
          ctx_ref, s1_ref, r1_ref, s2_ref, r2_ref, send_sems, recv_sems):
    my_i = lax.axis_index("i")
    p1 = my_i ^ 1
    p2 = 3 - my_i

    barrier_sem = pltpu.get_barrier_semaphore()
    for nbr in (p1, p2):
        pl.semaphore_signal(
            barrier_sem, inc=1,
            device_id=(nbr,), device_id_type=pl.DeviceIdType.MESH,
        )
    pl.semaphore_wait(barrier_sem, 2)

    xb = x_ref[...].astype(jnp.bfloat16)
    wqb = wq_ref[...].astype(jnp.bfloat16)
    qf = jnp.dot(xb, wqb, preferred_element_type=jnp.float32)
    qfb = qf.astype(jnp.bfloat16)
    kb = kf_ref[...].astype(jnp.bfloat16)
    vb = vf_ref[...].astype(jnp.bfloat16)
    wob = wo_ref[...].astype(jnp.bfloat16)

    r1 = []
    for b in range(B):
        rs = pl.ds(b * Sq, Sq)
        r0 = b * Sq
        for h in range(HL):
            cs = slice(h * Dh, (h + 1) * Dh)
            k = kb[r0:r0 + Skv, cs]
            v = vb[r0:r0 + Skv, cs]
            ctx_ref[0:BLK, cs] = _attn_block(
                qfb[r0:r0 + BLK, cs], k[0:BLK], v[0:BLK]
            ).astype(jnp.bfloat16)
            ctx_ref[BLK:Sq, cs] = _attn_block(
                qfb[r0 + BLK:r0 + Sq, cs], k, v
            ).astype(jnp.bfloat16)
        partial = jnp.dot(ctx_ref[...], wob,
                          preferred_element_type=jnp.float32)
        out_ref[rs, :] = partial
        s1_ref[b] = partial.astype(jnp.bfloat16)
        rdma = pltpu.make_async_remote_copy(
            src_ref=s1_ref.at[b],
            dst_ref=r1_ref.at[b],
            send_sem=send_sems.at[0, b],
            recv_sem=recv_sems.at[0, b],
            device_id=(p1,),
            device_id_type=pl.DeviceIdType.MESH,
        )
        rdma.start()
        r1.append(rdma)

    r2 = []
    for b in range(B):
        rs = pl.ds(b * Sq, Sq)
        r1[b].wait()
        pair = out_ref[rs, :] + r1_ref[b].astype(jnp.float32)
        out_ref[rs, :] = pair
        s2_ref[b] = pair.astype(jnp.bfloat16)
        rdma = pltpu.make_async_remote_copy(
            src_ref=s2_ref.at[b],
            dst_ref=r2_ref.at[b],
            send_sem=send_sems.at[1, b],
            recv_sem=recv_sems.at[1, b],
            device_id=(p2,),
            device_id_type=pl.DeviceIdType.MESH,
        )
        rdma.start()
        r2.append(rdma)

    for b in range(B):
        rs = pl.ds(b * Sq, Sq)
        r2[b].wait()
        out_ref[rs, :] = out_ref[rs, :] + r2_ref[b].astype(jnp.float32)


def kernel(x, Wq, K_ext, V_ext, Wo):
    my_i = lax.axis_index("i")
    Kh = lax.dynamic_slice_in_dim(K_ext, my_i * HL, HL, axis=2)
    Vh = lax.dynamic_slice_in_dim(V_ext, my_i * HL, HL, axis=2)

    out = pl.pallas_call(
        _body,
        out_shape=jax.ShapeDtypeStruct((ROWS, D_MODEL), jnp.float32),
        in_specs=[pl.BlockSpec(memory_space=pltpu.VMEM)] * 5,
        out_specs=pl.BlockSpec(memory_space=pltpu.VMEM),
        scratch_shapes=[
            pltpu.VMEM((Sq, D_HEADS), jnp.bfloat16),
            pltpu.VMEM((B, Sq, D_MODEL), jnp.bfloat16),
            pltpu.VMEM((B, Sq, D_MODEL), jnp.bfloat16),
            pltpu.VMEM((B, Sq, D_MODEL), jnp.bfloat16),
            pltpu.VMEM((B, Sq, D_MODEL), jnp.bfloat16),
            pltpu.SemaphoreType.DMA((2, B)),
            pltpu.SemaphoreType.DMA((2, B)),
        ],
        compiler_params=pltpu.CompilerParams(collective_id=0),
    )(
        x.reshape(ROWS, D_MODEL),
        Wq,
        Kh.reshape(B * Skv, D_HEADS),
        Vh.reshape(B * Skv, D_HEADS),
        Wo,
    )
    return out.reshape(B, Sq, D_MODEL)
